# baseline (speedup 1.0000x reference)
import jax
import jax.numpy as jnp
from jax import lax
from jax.experimental import pallas as pl
from jax.experimental.pallas import tpu as pltpu

_EPS_BN = 1e-5
_EPS_LN = 1e-5
_TEMP = 2.0
_ENT_LAMBDA = 0.1
_F32 = jnp.float32


def _teacher_body(x_ref, mean_ref, hsq_ref):
    x = x_ref[...]
    mean_ref[...] = jnp.mean(x, axis=2)
    hsq_ref[...] = jnp.mean(x * x, axis=1)


def _teacher_reduce(ft, bb):
    bs, c, h, w = ft.shape
    hw = h * w
    x = ft.reshape(bs, c, hw)
    return pl.pallas_call(
        _teacher_body,
        grid=(bs // bb,),
        in_specs=[pl.BlockSpec((bb, c, hw), lambda i: (i, 0, 0))],
        out_specs=[pl.BlockSpec((bb, c), lambda i: (i, 0)),
                   pl.BlockSpec((bb, hw), lambda i: (i, 0))],
        out_shape=[jax.ShapeDtypeStruct((bs, c), _F32),
                   jax.ShapeDtypeStruct((bs, hw), _F32)],
        compiler_params=pltpu.CompilerParams(
            dimension_semantics=("parallel",),
            vmem_limit_bytes=48 * 1024 * 1024),
        name="teacher_reduce",
    )(x)


def _student_body(*refs):
    xs = refs[:16]
    cm_ref, p_ref = refs[16], refs[17]
    for s in range(16):
        x = xs[s][...]
        cm_ref[:, s, :] = 0.7 * jnp.mean(x, axis=2) + 0.3 * jnp.max(x, axis=2)
        p_ref[:, s, :] = jnp.mean(x * x, axis=1)


def _student_reduce(g_s, bb):
    bs = g_s[0].shape[0]
    xs = [g.reshape(bs, 16, 1024) for g in g_s]
    spec = pl.BlockSpec((bb, 16, 1024), lambda i: (i, 0, 0))
    return pl.pallas_call(
        _student_body,
        grid=(bs // bb,),
        in_specs=[spec] * 16,
        out_specs=[pl.BlockSpec((bb, 16, 16), lambda i: (i, 0, 0)),
                   pl.BlockSpec((bb, 16, 1024), lambda i: (i, 0, 0))],
        out_shape=[jax.ShapeDtypeStruct((bs, 16, 16), _F32),
                   jax.ShapeDtypeStruct((bs, 16, 1024), _F32)],
        compiler_params=pltpu.CompilerParams(
            dimension_semantics=("parallel",),
            vmem_limit_bytes=48 * 1024 * 1024),
        name="student_reduce",
    )(*xs)


def _bn_batch(x, g, b):
    mu = jnp.mean(x, axis=0, keepdims=True)
    xc = x - mu
    v = jnp.mean(xc * xc, axis=0, keepdims=True)
    return xc * lax.rsqrt(v + _EPS_BN) * g + b


def _l2n(x):
    n = jnp.sqrt(jnp.sum(x * x, axis=1, keepdims=True))
    return x / jnp.maximum(n, 1e-12)


def _sig(x):
    return 1.0 / (1.0 + jnp.exp(-x))


def _epilogue_body(cm_ref, p_ref, q2_ref, q4_ref, q8_ref,
                   tm0_ref, tm1_ref, tm2_ref, tm3_ref,
                   hs0_ref, hs1_ref, hs2_ref, hs3_ref,
                   aw_ref, ab_ref,
                   kW_ref, kb_ref, kg_ref, kbeta_ref,
                   W1_ref, b1_ref, g1_ref, beta1_ref,
                   W2_ref, b2_ref, g2_ref, beta2_ref,
                   qW0_ref, qW1_ref, qW2_ref, qW3_ref,
                   qb_ref, qg_ref, qbeta_ref,
                   pt_ref, ps_ref, lw_ref,
                   lng0_ref, lng1_ref, lng2_ref, lng3_ref,
                   lnb0_ref, lnb1_ref, lnb2_ref, lnb3_ref,
                   out_ref):
    cn = (((1,), (1,)), ((), ()))

    cm = cm_ref[...]
    ks = []
    for s in range(16):
        k_s = lax.dot_general(cm[:, s, :], kW_ref[s], cn,
                              preferred_element_type=_F32) + kb_ref[s]
        k_s = jnp.maximum(_bn_batch(k_s, kg_ref[s], kbeta_ref[s]), 0.0)
        ks.append(k_s[:, None, :])
    key2 = jnp.concatenate(ks, axis=1).reshape(1024, 128)

    h1 = lax.dot_general(key2, W1_ref[...], cn,
                         preferred_element_type=_F32) + b1_ref[...]
    h1 = jnp.maximum(_bn_batch(h1, g1_ref[...], beta1_ref[...]), 0.0)
    h2 = lax.dot_general(h1, W2_ref[...], cn,
                         preferred_element_type=_F32) + b2_ref[...]
    h2 = jnp.maximum(_bn_batch(h2, g2_ref[...], beta2_ref[...]), 0.0)

    tms = (tm0_ref, tm1_ref, tm2_ref, tm3_ref)
    qWs = (qW0_ref, qW1_ref, qW2_ref, qW3_ref)
    nqs = []
    for t in range(4):
        q = lax.dot_general(tms[t][...], qWs[t][...], cn,
                            preferred_element_type=_F32) + qb_ref[t]
        q = _bn_batch(q, qg_ref[t], qbeta_ref[t])
        nqs.append(_l2n(q))

    pp = lax.dot_general(pt_ref[...], ps_ref[...], cn,
                         preferred_element_type=_F32)
    atts = []
    ent_acc = jnp.zeros((1, 1), _F32)
    for t in range(4):
        nk = _l2n(h2[:, t * 128:(t + 1) * 128])
        cos = jnp.sum(nk.reshape(64, 16, 128) * nqs[t][:, None, :], axis=2)
        logit = (cos + pp[t]) * (1.0 / _TEMP)
        m = jnp.max(logit, axis=1, keepdims=True)
        e = jnp.exp(logit - m)
        att = e / jnp.sum(e, axis=1, keepdims=True)
        atts.append(att)
        ent_acc = ent_acc + jnp.sum(att * jnp.log(att + 1e-8), keepdims=True)
    total = _ENT_LAMBDA * (-ent_acc / 256.0)

    lwv = lw_ref[...]
    le = jnp.exp(lwv - jnp.max(lwv, axis=1, keepdims=True))
    wts = le / jnp.sum(le, axis=1, keepdims=True)

    p2d = p_ref[...]
    pm = jnp.mean(p2d, axis=1, keepdims=True)

    combs = [_sig(aw_ref[0, 0] * pm + ab_ref[0, 0]) * p2d]

    m2 = 0.25 * (q2_ref[0] + q2_ref[1] + q2_ref[2] + q2_ref[3])
    x2 = jnp.maximum(jnp.maximum(q2_ref[0], q2_ref[1]),
                     jnp.maximum(q2_ref[2], q2_ref[3]))
    combs.append(_sig(aw_ref[0, 1] * pm + ab_ref[0, 1]) * (0.7 * m2 + 0.3 * x2))

    s_acc = q4_ref[0]
    x_acc = q4_ref[0]
    for j in range(1, 16):
        xj = q4_ref[j]
        s_acc = s_acc + xj
        x_acc = jnp.maximum(x_acc, xj)
    combs.append(_sig(aw_ref[0, 2] * pm + ab_ref[0, 2])
                 * (0.7 / 16.0 * s_acc + 0.3 * x_acc))

    q8 = q8_ref[...]
    m8 = jnp.mean(q8, axis=2)
    x8 = jnp.max(q8, axis=2)
    combs.append(_sig(aw_ref[0, 3] * pm + ab_ref[0, 3]) * (0.7 * m8 + 0.3 * x8))

    lngs = (lng0_ref, lng1_ref, lng2_ref, lng3_ref)
    lnbs = (lnb0_ref, lnb1_ref, lnb2_ref, lnb3_ref)
    hts = (hs0_ref, hs1_ref, hs2_ref, hs3_ref)
    for t in range(4):
        val = _l2n(combs[t])
        hw = val.shape[1]
        v3 = val.reshape(64, 16, hw)
        mu = jnp.mean(v3, axis=1, keepdims=True)
        xc = v3 - mu
        var = jnp.mean(xc * xc, axis=1, keepdims=True)
        hn = xc * lax.rsqrt(var + _EPS_LN) * lngs[t][...][None] + lnbs[t][...][None]
        htn = _l2n(hts[t][...])
        d = hn - htn[:, None, :]
        ad = jnp.abs(d)
        sm = jnp.mean(jnp.where(ad < 1.0, 0.5 * d * d, ad - 0.5), axis=2)
        mse = jnp.mean(d * d, axis=2)
        diff = 0.7 * sm + 0.3 * mse * mse
        total = total + (jnp.sum(diff * atts[t], keepdims=True) / 64.0) * wts[:, t:t + 1]

    out_ref[...] = total


def _epilogue(cm, p2d, q2, q4, q8, tms, hsqs, params):
    n_vmem = 13 + len(params) - 2
    in_specs = ([pl.BlockSpec(memory_space=pltpu.VMEM)] * 13
                + [pl.BlockSpec(memory_space=pltpu.SMEM)] * 2
                + [pl.BlockSpec(memory_space=pltpu.VMEM)] * (len(params) - 2))
    return pl.pallas_call(
        _epilogue_body,
        in_specs=in_specs,
        out_specs=pl.BlockSpec(memory_space=pltpu.VMEM),
        out_shape=jax.ShapeDtypeStruct((1, 1), _F32),
        compiler_params=pltpu.CompilerParams(
            vmem_limit_bytes=50 * 1024 * 1024),
        name="afd_epilogue",
    )(cm, p2d, q2, q4, q8, *tms, *hsqs, *params)


def kernel(g_s_0, g_s_1, g_s_2, g_s_3, g_s_4, g_s_5, g_s_6, g_s_7,
           g_s_8, g_s_9, g_s_10, g_s_11, g_s_12, g_s_13, g_s_14, g_s_15,
           g_t_0, g_t_1, g_t_2, g_t_3,
           attn_w, attn_b, key_W, key_b, key_g, key_beta,
           W1, b1, g1, beta1, W2, b2, g2, beta2,
           q_W0, q_W1, q_W2, q_W3, q_b, q_g, q_beta,
           p_t, p_s, layer_weights, ln_g, ln_b):
    g_s = [g_s_0, g_s_1, g_s_2, g_s_3, g_s_4, g_s_5, g_s_6, g_s_7,
           g_s_8, g_s_9, g_s_10, g_s_11, g_s_12, g_s_13, g_s_14, g_s_15]
    g_t = [g_t_0, g_t_1, g_t_2, g_t_3]

    t_bb = (8, 8, 8, 8)
    tms, hsqs = [], []
    for i in range(4):
        tm, hsq = _teacher_reduce(g_t[i], t_bb[i])
        tms.append(tm)
        hsqs.append(hsq)

    cm, p = _student_reduce(g_s, bb=8)

    q2 = p.reshape(64, 16, 16, 2, 16, 2).transpose(3, 5, 0, 1, 2, 4) \
          .reshape(4, 1024, 256)
    q4 = p.reshape(64, 16, 8, 4, 8, 4).transpose(3, 5, 0, 1, 2, 4) \
          .reshape(16, 1024, 64)
    q8 = p.reshape(64, 16, 4, 8, 4, 8).transpose(0, 1, 2, 4, 3, 5) \
          .reshape(1024, 16, 64)
    p2d = p.reshape(1024, 1024)

    hws = (1024, 256, 64, 16)
    params = [attn_w.reshape(1, 4), attn_b.reshape(1, 4),
              key_W, key_b, key_g, key_beta,
              W1, b1.reshape(1, 256), g1.reshape(1, 256), beta1.reshape(1, 256),
              W2, b2.reshape(1, 512), g2.reshape(1, 512), beta2.reshape(1, 512),
              q_W0, q_W1, q_W2, q_W3, q_b, q_g, q_beta,
              p_t, p_s, layer_weights.reshape(1, 4)]
    params += [jnp.broadcast_to(ln_g[t, :, None], (16, hws[t])) for t in range(4)]
    params += [jnp.broadcast_to(ln_b[t, :, None], (16, hws[t])) for t in range(4)]

    out = _epilogue(cm, p2d, q2, q4, q8, tms, hsqs, params)
    return out.reshape(())
